# baseline (speedup 1.0000x reference)
import jax
from jax.experimental import pallas as pl
from jax.experimental.pallas import tpu as pltpu


def _copy_block(x_ref, o_ref):
    o_ref[...] = x_ref[...]


def kernel(x):
    b, c, h, w = x.shape
    blk_c = c
    for cand in range(c, 0, -1):
        if c % cand == 0 and cand * h * w * x.dtype.itemsize <= 3 * 1024 * 1024:
            blk_c = cand
            break
    grid = (b, c // blk_c)
    return pl.pallas_call(
        _copy_block,
        out_shape=jax.ShapeDtypeStruct(x.shape, x.dtype),
        grid=grid,
        in_specs=[pl.BlockSpec((1, blk_c, h, w), lambda i, j: (i, j, 0, 0))],
        out_specs=pl.BlockSpec((1, blk_c, h, w), lambda i, j: (i, j, 0, 0)),
        compiler_params=pltpu.CompilerParams(
            dimension_semantics=("parallel", "parallel")),
    )(x)


# speedup vs baseline: 1.0533x; 1.0533x over previous
import jax
from jax.experimental import pallas as pl
from jax.experimental.pallas import tpu as pltpu


def _copy_block(x_ref, o_ref):
    o_ref[...] = x_ref[...]


def kernel(x):
    b, c, h, w = x.shape
    blk_c = c
    for cand in range(c, 0, -1):
        if c % cand == 0 and cand * h * w * x.dtype.itemsize <= 10 * 1024 * 1024:
            blk_c = cand
            break
    grid = (b, c // blk_c)
    return pl.pallas_call(
        _copy_block,
        out_shape=jax.ShapeDtypeStruct(x.shape, x.dtype),
        grid=grid,
        in_specs=[pl.BlockSpec((1, blk_c, h, w), lambda i, j: (i, j, 0, 0))],
        out_specs=pl.BlockSpec((1, blk_c, h, w), lambda i, j: (i, j, 0, 0)),
        compiler_params=pltpu.CompilerParams(
            dimension_semantics=("parallel", "parallel")),
    )(x)
